# Initial kernel scaffold; baseline (speedup 1.0000x reference)
import functools

import jax
import jax.numpy as jnp
from jax import lax
from jax.experimental import pallas as pl
from jax.experimental.pallas import tpu as pltpu
from jax.experimental.pallas import tpu_sc as plsc

B, C, N_IN = 4, 64, 8192
K, P = 4, 4
N_OUT = 32768
ROWS = B * C
NW = 32
ROWS_PER_W = ROWS // NW
CHUNK = 2048
CHUNK_WORDS = CHUNK * K * P
N_CHUNKS = N_IN // CHUNK
L = 16


def _unpool_body(x_hbm, idx_hbm, sm_hbm, iw_hbm, out_hbm,
                 acc, xr, ir, smc, iwc):
    nc = 2
    wid = lax.axis_index("s") * nc + lax.axis_index("c")
    lane = jnp.arange(L, dtype=jnp.int32)
    zero = jnp.zeros((L,), jnp.float32)

    def row_body(i, _):
        r = wid * ROWS_PER_W + i

        def zbody(j, _):
            acc[pl.ds(j * L, L)] = zero
            return 0

        lax.fori_loop(0, N_OUT // L, zbody, 0)
        pltpu.sync_copy(x_hbm.at[r], xr)
        pltpu.sync_copy(idx_hbm.at[r], ir)

        def chunk_body(c, _):
            pltpu.sync_copy(sm_hbm.at[pl.ds(c * CHUNK_WORDS, CHUNK_WORDS)], smc)
            pltpu.sync_copy(iw_hbm.at[pl.ds(c * CHUNK_WORDS, CHUNK_WORDS)], iwc)

            def inner(t, _):
                n0 = t * L
                kv = ir[pl.ds(c * CHUNK + n0, L)]
                xv = xr[pl.ds(c * CHUNK + n0, L)]
                addr = (lane + n0) * (K * P) + kv * P
                for p in range(P):
                    ap = addr + p
                    smv = plsc.load_gather(smc, [ap])
                    iwv = plsc.load_gather(iwc, [ap])
                    plsc.addupdate_scatter(acc, [smv], xv * iwv)
                return 0

            lax.fori_loop(0, CHUNK // L, inner, 0)
            return 0

        lax.fori_loop(0, N_CHUNKS, chunk_body, 0)
        pltpu.sync_copy(acc, out_hbm.at[r])
        return 0

    lax.fori_loop(0, ROWS_PER_W, row_body, 0)


@jax.jit
def _unpool(xf, idxf, smf, iwf):
    mesh = plsc.VectorSubcoreMesh(core_axis_name="c", subcore_axis_name="s")
    f = functools.partial(
        pl.kernel,
        mesh=mesh,
        out_type=jax.ShapeDtypeStruct((ROWS, N_OUT), jnp.float32),
        scratch_types=[
            pltpu.VMEM((N_OUT,), jnp.float32),
            pltpu.VMEM((N_IN,), jnp.float32),
            pltpu.VMEM((N_IN,), jnp.int32),
            pltpu.VMEM((CHUNK_WORDS,), jnp.int32),
            pltpu.VMEM((CHUNK_WORDS,), jnp.float32),
        ],
    )(_unpool_body)
    return f(xf, idxf, smf, iwf)


def kernel(x, idx_mask, sample_map, interp_weights):
    xf = x.reshape(ROWS, N_IN)
    idxf = idx_mask.reshape(ROWS, N_IN).astype(jnp.int32)
    smf = sample_map.reshape(-1).astype(jnp.int32)
    iwf = interp_weights.reshape(-1)
    out = _unpool(xf, idxf, smf, iwf)
    return out.reshape(B, C, N_OUT)


# speedup vs baseline: 180.2369x; 180.2369x over previous
import functools

import jax
import jax.numpy as jnp
from jax import lax
from jax.experimental import pallas as pl
from jax.experimental.pallas import tpu as pltpu
from jax.experimental.pallas import tpu_sc as plsc

B, C, N_IN = 4, 64, 8192
K, P = 4, 4
N_OUT = 32768
ROWS = B * C
NW = 32
ROWS_PER_W = ROWS // NW
CHUNK = 2048
CHUNK_WORDS = CHUNK * K * P
N_CHUNKS = N_IN // CHUNK
L = 16


def _unpool_body(x_hbm, idx_hbm, sm_hbm, iw_hbm, out_hbm,
                 acc, xr, ir, smc, iwc):
    nc = 2
    wid = lax.axis_index("s") * nc + lax.axis_index("c")
    lane = jnp.arange(L, dtype=jnp.int32)
    zero = jnp.zeros((L,), jnp.float32)

    def row_body(i, _):
        r = wid * ROWS_PER_W + i

        def zbody(j, _):
            acc[pl.ds(j * L, L)] = zero
            return 0

        lax.fori_loop(0, N_OUT // L, zbody, 0)
        pltpu.sync_copy(x_hbm.at[r], xr)
        pltpu.sync_copy(idx_hbm.at[r], ir)

        def chunk_body(c, _):
            pltpu.sync_copy(sm_hbm.at[pl.ds(c * CHUNK_WORDS, CHUNK_WORDS)], smc)
            pltpu.sync_copy(iw_hbm.at[pl.ds(c * CHUNK_WORDS, CHUNK_WORDS)], iwc)

            def inner(t, _):
                n0 = t * L
                kv = ir[pl.ds(c * CHUNK + n0, L)]
                xv = xr[pl.ds(c * CHUNK + n0, L)]
                addr = (lane + n0) * (K * P) + kv * P
                for p in range(P):
                    ap = addr + p
                    smv = plsc.load_gather(smc, [ap])
                    iwv = plsc.load_gather(iwc, [ap])
                    plsc.addupdate_scatter(acc, [smv], xv * iwv)
                return 0

            lax.fori_loop(0, CHUNK // L, inner, 0)
            return 0

        lax.fori_loop(0, N_CHUNKS, chunk_body, 0)
        pltpu.sync_copy(acc, out_hbm.at[r])
        return 0

    lax.fori_loop(0, ROWS_PER_W, row_body, 0)


@jax.jit
def _unpool(xf, idxf, smf, iwf):
    mesh = plsc.VectorSubcoreMesh(core_axis_name="c", subcore_axis_name="s")
    f = functools.partial(
        pl.kernel,
        mesh=mesh,
        compiler_params=pltpu.CompilerParams(needs_layout_passes=False),
        out_type=jax.ShapeDtypeStruct((ROWS, N_OUT), jnp.float32),
        scratch_types=[
            pltpu.VMEM((N_OUT,), jnp.float32),
            pltpu.VMEM((N_IN,), jnp.float32),
            pltpu.VMEM((N_IN,), jnp.int32),
            pltpu.VMEM((CHUNK_WORDS,), jnp.int32),
            pltpu.VMEM((CHUNK_WORDS,), jnp.float32),
        ],
    )(_unpool_body)
    return f(xf, idxf, smf, iwf)


def kernel(x, idx_mask, sample_map, interp_weights):
    xf = x.reshape(ROWS, N_IN)
    idxf = idx_mask.reshape(ROWS, N_IN).astype(jnp.int32)
    smf = sample_map.reshape(-1).astype(jnp.int32)
    iwf = interp_weights.reshape(-1)
    out = _unpool(xf, idxf, smf, iwf)
    return out.reshape(B, C, N_OUT)


# speedup vs baseline: 257.2806x; 1.4275x over previous
import functools

import jax
import jax.numpy as jnp
from jax import lax
from jax.experimental import pallas as pl
from jax.experimental.pallas import tpu as pltpu
from jax.experimental.pallas import tpu_sc as plsc

B, C, N_IN = 4, 64, 8192
K, P = 4, 4
N_OUT = 32768
ROWS = B * C
NW = 32
ROWS_PER_W = ROWS // NW
NKROWS = N_IN * K
TW = 2 * P
L = 16
Q = 2048
NQ = N_IN // Q


def _unpool_body(x_hbm, idxq_hbm, smiw_hbm, out_hbm,
                 acc, xr, irq, gselq, smiw_sp, semA, semB):
    nc = 2
    wid = lax.axis_index("s") * nc + lax.axis_index("c")
    lane = jnp.arange(L, dtype=jnp.int32)
    zero = jnp.zeros((L,), jnp.float32)
    sems = [semA, semB]

    @pl.when(lax.axis_index("s") == 0)
    def _():
        pltpu.sync_copy(smiw_hbm, smiw_sp)

    plsc.subcore_barrier()

    def compute_quarter(q, buf):
        def inner(t, _):
            n0 = t * L
            xv = xr[pl.ds(q * Q + n0, L)]
            rows = lane + n0
            for p in range(P):
                smv = plsc.load_gather(
                    gselq, [jnp.full((L,), buf, jnp.int32), rows,
                            jnp.full((L,), p, jnp.int32)])
                iwv = plsc.bitcast(
                    plsc.load_gather(
                        gselq, [jnp.full((L,), buf, jnp.int32), rows,
                                jnp.full((L,), P + p, jnp.int32)]),
                    jnp.float32)
                plsc.addupdate_scatter(acc, [smv], xv * iwv)
            return 0

        lax.fori_loop(0, Q // L, inner, 0)

    def row_body(i, _):
        r = wid * ROWS_PER_W + i
        pltpu.sync_copy(x_hbm.at[r], xr)

        copies = [None, None]
        for q in range(NQ):
            buf = q % 2
            pltpu.sync_copy(idxq_hbm.at[r * NQ + q], irq.at[buf])

            def gix_body(t, _):
                n0 = t * L
                kv = irq[buf, pl.ds(n0, L)]
                irq[buf, pl.ds(n0, L)] = (lane + (q * Q + n0)) * K + kv
                return 0

            lax.fori_loop(0, Q // L, gix_body, 0)
            cp = pltpu.make_async_copy(
                smiw_sp.at[irq.at[buf]], gselq.at[buf], sems[buf])
            cp.start()
            copies[buf] = cp
            if q == 0:
                def zbody(j, _):
                    base = j * (L * 8)
                    for u in range(8):
                        acc[pl.ds(base + u * L, L)] = zero
                    return 0

                lax.fori_loop(0, N_OUT // (L * 8), zbody, 0)
            else:
                copies[1 - buf].wait()
                compute_quarter(q - 1, 1 - buf)
        copies[(NQ - 1) % 2].wait()
        compute_quarter(NQ - 1, (NQ - 1) % 2)
        pltpu.sync_copy(acc, out_hbm.at[r])
        return 0

    lax.fori_loop(0, ROWS_PER_W, row_body, 0)


@jax.jit
def _unpool(xf, idxq, smiwf):
    mesh = plsc.VectorSubcoreMesh(core_axis_name="c", subcore_axis_name="s")
    f = functools.partial(
        pl.kernel,
        mesh=mesh,
        compiler_params=pltpu.CompilerParams(
            needs_layout_passes=False, use_tc_tiling_on_sc=False),
        out_type=jax.ShapeDtypeStruct((ROWS, N_OUT), jnp.float32),
        scratch_types=[
            pltpu.VMEM((N_OUT,), jnp.float32),
            pltpu.VMEM((N_IN,), jnp.float32),
            pltpu.VMEM((2, Q), jnp.int32),
            pltpu.VMEM((2, Q, TW), jnp.int32),
            pltpu.VMEM_SHARED((NKROWS, TW), jnp.int32),
            pltpu.SemaphoreType.DMA,
            pltpu.SemaphoreType.DMA,
        ],
    )(_unpool_body)
    return f(xf, idxq, smiwf)


def kernel(x, idx_mask, sample_map, interp_weights):
    xf = x.reshape(ROWS, N_IN)
    idxq = idx_mask.reshape(ROWS * NQ, Q).astype(jnp.int32)
    smf = sample_map.reshape(NKROWS, P).astype(jnp.int32)
    iwf = lax.bitcast_convert_type(
        interp_weights.reshape(NKROWS, P), jnp.int32)
    smiwf = jnp.concatenate([smf, iwf], axis=1)
    out = _unpool(xf, idxq, smiwf)
    return out.reshape(B, C, N_OUT)
